# baseline (speedup 1.0000x reference)
import functools

import jax
import jax.numpy as jnp
from jax import lax
from jax.experimental import pallas as pl
from jax.experimental.pallas import tpu as pltpu
from jax.experimental.pallas import tpu_sc as plsc

N = 10000
E = 320000
D_FEAT = 128
D = 64
NUM_ACTIONS = 10

NC = 2
NS = 16
NW = NC * NS
C = 128
EPW = E // NW
CHUNKS = -(-EPW // C)
EPWP = CHUNKS * C
NB = 8
NP = 10240
RPS = NP // NS
DEGW = 16


def _degree_body(dst_hbm, ones_hbm, zeros_hbm, out_hbm, didx, ones_v, acc, sem):
    c = lax.axis_index("c")
    s = lax.axis_index("s")
    w = c * NS + s
    pltpu.sync_copy(zeros_hbm, acc.at[pl.ds(s * RPS, RPS)])
    pltpu.sync_copy(dst_hbm.at[w], didx)
    pltpu.sync_copy(ones_hbm, ones_v)
    plsc.subcore_barrier()

    def fire(k, carry):
        pltpu.async_copy(ones_v, acc.at[didx.at[k]], sem, add=True)
        return carry

    lax.fori_loop(0, CHUNKS, fire, 0)

    def drain(k, carry):
        pltpu.make_async_copy(ones_v, acc.at[didx.at[k]], sem).wait()
        return carry

    lax.fori_loop(0, CHUNKS, drain, 0)
    plsc.subcore_barrier()
    pltpu.sync_copy(acc.at[pl.ds(s * RPS, RPS)],
                    out_hbm.at[c, pl.ds(s * RPS, RPS)])


def _gs_body(src_hbm, dst_hbm, z_hbm, zeros_hbm, out_hbm,
             sidx, didx, acc, *bufs):
    rows = bufs[:NB]
    gsem = bufs[NB:2 * NB]
    ssem = bufs[2 * NB:3 * NB]
    c = lax.axis_index("c")
    s = lax.axis_index("s")
    w = c * NS + s
    pltpu.sync_copy(zeros_hbm, acc.at[pl.ds(s * RPS, RPS)])
    pltpu.sync_copy(src_hbm.at[w], sidx)
    pltpu.sync_copy(dst_hbm.at[w], didx)
    plsc.subcore_barrier()

    def gather(k, b):
        pltpu.async_copy(z_hbm.at[sidx.at[k]], rows[b], gsem[b])

    def gather_wait(k, b):
        pltpu.make_async_copy(z_hbm.at[sidx.at[k]], rows[b], gsem[b]).wait()

    def scatter(k, b):
        pltpu.async_copy(rows[b], acc.at[didx.at[k]], ssem[b], add=True)

    def scatter_wait(k, b):
        pltpu.make_async_copy(rows[b], acc.at[didx.at[k]], ssem[b]).wait()

    GA = NB - 3

    for b in range(GA):
        gather(b, b)
    for b in range(NB):
        k = b
        gather_wait(k, b)
        scatter(k, b)
        bq = (b + GA) % NB
        if k >= 3:
            scatter_wait(k - 3, bq)
        gather(k + GA, bq)

    def block(j, carry):
        for b in range(NB):
            k = j * NB + b
            gather_wait(k, b)
            scatter(k, b)
            bq = (b + GA) % NB
            scatter_wait(k - 3, bq)
            gather(k + GA, bq)
        return carry

    nfull = CHUNKS // NB
    lax.fori_loop(1, nfull, block, 0)
    for k in range(nfull * NB, CHUNKS):
        b = k % NB
        gather_wait(k, b)
        scatter(k, b)
        scatter_wait(k - 3, (b + GA) % NB)
        if k + GA < CHUNKS:
            gather(k + GA, (b + GA) % NB)
    for k in range(CHUNKS - 3, CHUNKS):
        scatter_wait(k, k % NB)

    plsc.subcore_barrier()
    pltpu.sync_copy(acc.at[pl.ds(s * RPS, RPS)],
                    out_hbm.at[c, pl.ds(s * RPS, RPS)])


@functools.cache
def _sc_kernels():
    mesh = plsc.VectorSubcoreMesh(core_axis_name="c", subcore_axis_name="s",
                                  num_cores=NC, num_subcores=NS)
    sc_degree = functools.partial(
        pl.kernel,
        out_type=jax.ShapeDtypeStruct((NC, NP, DEGW), jnp.float32),
        mesh=mesh,
        compiler_params=pltpu.CompilerParams(use_tc_tiling_on_sc=False),
        scratch_types=[
            pltpu.VMEM((CHUNKS, C), jnp.int32),
            pltpu.VMEM((C, DEGW), jnp.float32),
            pltpu.VMEM_SHARED((NP, DEGW), jnp.float32),
            pltpu.SemaphoreType.DMA,
        ],
    )(_degree_body)
    sc_gather_scatter = functools.partial(
        pl.kernel,
        out_type=jax.ShapeDtypeStruct((NC, NP, D), jnp.float32),
        mesh=mesh,
        compiler_params=pltpu.CompilerParams(use_tc_tiling_on_sc=False),
        scratch_types=(
            [pltpu.VMEM((CHUNKS, C), jnp.int32),
             pltpu.VMEM((CHUNKS, C), jnp.int32),
             pltpu.VMEM_SHARED((NP, D), jnp.float32)]
            + [pltpu.VMEM((C, D), jnp.float32) for _ in range(NB)]
            + [pltpu.SemaphoreType.DMA for _ in range(2 * NB)]
        ),
    )(_gs_body)
    return sc_degree, sc_gather_scatter


R = 2000


def _bdot(a, b):
    return jnp.dot(a, b, preferred_element_type=jnp.float32)


def _hdot(a, b):
    return jnp.dot(a, b, preferred_element_type=jnp.float32,
                   precision=lax.Precision.HIGHEST)


def _dinv_block(deg_ref):
    deg = 1.0 + deg_ref[0, :, 0:1] + deg_ref[1, :, 0:1]
    return 1.0 / jnp.sqrt(deg)


def _mm1_kernel(nodes_ref, wfc_ref, bfc_ref, w1_ref, deg_ref, z1_ref):
    h = jnp.maximum(_bdot(nodes_ref[...], wfc_ref[...]) + bfc_ref[...], 0.0)
    z1_ref[...] = _bdot(h, w1_ref[...]) * _dinv_block(deg_ref)


def _mm2_kernel(a_ref, z1_ref, deg_ref, b1_ref, w2_ref, z2_ref):
    dinv = _dinv_block(deg_ref)
    h1 = jnp.maximum(
        (a_ref[0] + a_ref[1] + z1_ref[...]) * dinv + b1_ref[...], 0.0)
    z2_ref[...] = _bdot(h1, w2_ref[...]) * dinv


def _fin_kernel(a_ref, z2_ref, deg_ref, b2_ref, wa_ref, ba_ref, wc_ref,
                bc_ref, logits_ref, value_ref, sum_ref):
    i = pl.program_id(0)
    dinv = _dinv_block(deg_ref)
    h2 = jnp.maximum(
        (a_ref[0] + a_ref[1] + z2_ref[...]) * dinv + b2_ref[...], 0.0)
    part = jnp.sum(h2, axis=0, keepdims=True)

    @pl.when(i == 0)
    def _init():
        sum_ref[...] = jnp.zeros_like(sum_ref)

    sum_ref[...] += part

    @pl.when(i == pl.num_programs(0) - 1)
    def _finish():
        rep = sum_ref[...] * (1.0 / N)
        logits_ref[...] = _hdot(rep, wa_ref[...]) + ba_ref[...]
        value_ref[...] = _hdot(rep, wc_ref[...]) + bc_ref[...]


def _full(shape):
    return pl.BlockSpec(shape, lambda i: (0,) * len(shape))


@functools.cache
def _tc_kernels():
    mm1 = pl.pallas_call(
    _mm1_kernel,
    grid=(N // R,),
    in_specs=[
        pl.BlockSpec((R, D_FEAT), lambda i: (i, 0)),
        _full((D_FEAT, D)),
        _full((1, D)),
        _full((D, D)),
        pl.BlockSpec((NC, R, DEGW), lambda i: (0, i, 0)),
    ],
    out_specs=pl.BlockSpec((R, D), lambda i: (i, 0)),
    out_shape=jax.ShapeDtypeStruct((N, D), jnp.float32),
    )
    mm2 = pl.pallas_call(
    _mm2_kernel,
    grid=(N // R,),
    in_specs=[
        pl.BlockSpec((NC, R, D), lambda i: (0, i, 0)),
        pl.BlockSpec((R, D), lambda i: (i, 0)),
        pl.BlockSpec((NC, R, DEGW), lambda i: (0, i, 0)),
        _full((1, D)),
        _full((D, D)),
    ],
    out_specs=pl.BlockSpec((R, D), lambda i: (i, 0)),
    out_shape=jax.ShapeDtypeStruct((N, D), jnp.float32),
    )
    fin = pl.pallas_call(
    _fin_kernel,
    grid=(N // R,),
    in_specs=[
        pl.BlockSpec((NC, R, D), lambda i: (0, i, 0)),
        pl.BlockSpec((R, D), lambda i: (i, 0)),
        pl.BlockSpec((NC, R, DEGW), lambda i: (0, i, 0)),
        _full((1, D)),
        _full((D, NUM_ACTIONS)),
        _full((1, NUM_ACTIONS)),
        _full((D, 1)),
        _full((1, 1)),
    ],
    out_specs=[_full((1, NUM_ACTIONS)), _full((1, 1))],
    out_shape=[jax.ShapeDtypeStruct((1, NUM_ACTIONS), jnp.float32),
               jax.ShapeDtypeStruct((1, 1), jnp.float32)],
    scratch_shapes=[pltpu.VMEM((1, D), jnp.float32)],
    )
    return mm1, mm2, fin


def kernel(nodes, edge_links, W_fc, b_fc, W1, b1, W2, b2, Wa, ba, Wc, bc):
    src = edge_links[:, 0].reshape(NW, EPW)
    dst = edge_links[:, 1].reshape(NW, EPW)
    src = jnp.pad(src, ((0, 0), (0, EPWP - EPW))).reshape(NW, CHUNKS, C)
    dst = jnp.pad(dst, ((0, 0), (0, EPWP - EPW)),
                  constant_values=NP - 1).reshape(NW, CHUNKS, C)
    zeros_d = jnp.zeros((RPS, D), jnp.float32)
    zeros_g = jnp.zeros((RPS, DEGW), jnp.float32)
    ones_g = jnp.ones((C, DEGW), jnp.float32)

    sc_degree, sc_gather_scatter = _sc_kernels()
    mm1, mm2, fin = _tc_kernels()
    deg = sc_degree(dst, ones_g, zeros_g)
    z1 = mm1(nodes, W_fc, b_fc.reshape(1, D), W1, deg)
    a1 = sc_gather_scatter(src, dst, z1, zeros_d)
    z2 = mm2(a1, z1, deg, b1.reshape(1, D), W2)
    a2 = sc_gather_scatter(src, dst, z2, zeros_d)
    logits, value = fin(a2, z2, deg, b2.reshape(1, D), Wa,
                        ba.reshape(1, NUM_ACTIONS), Wc, bc.reshape(1, 1))
    return logits, value


# speedup vs baseline: 1.5747x; 1.5747x over previous
import functools

import jax
import jax.numpy as jnp
from jax import lax
from jax.experimental import pallas as pl
from jax.experimental.pallas import tpu as pltpu
from jax.experimental.pallas import tpu_sc as plsc

N = 10000
E = 320000
D_FEAT = 128
D = 64
NUM_ACTIONS = 10

NC = 2
NS = 16
NW = NC * NS
C = 80
EPW = E // NW
CHUNKS = EPW // C
NB = 8
NP = 10240
RPS = NP // NS
DEGW = 16


def _degree_body(dst_hbm, ones_hbm, zeros_hbm, out_hbm, didx, ones_v, acc, sem):
    c = lax.axis_index("c")
    s = lax.axis_index("s")
    w = c * NS + s
    pltpu.sync_copy(zeros_hbm, acc.at[pl.ds(s * RPS, RPS)])
    pltpu.sync_copy(dst_hbm.at[w], didx)
    pltpu.sync_copy(ones_hbm, ones_v)
    plsc.subcore_barrier()

    def fire(k, carry):
        pltpu.async_copy(ones_v, acc.at[didx.at[k]], sem, add=True)
        return carry

    lax.fori_loop(0, CHUNKS, fire, 0)

    def drain(k, carry):
        pltpu.make_async_copy(ones_v, acc.at[didx.at[k]], sem).wait()
        return carry

    lax.fori_loop(0, CHUNKS, drain, 0)
    plsc.subcore_barrier()
    pltpu.sync_copy(acc.at[pl.ds(s * RPS, RPS)],
                    out_hbm.at[c, pl.ds(s * RPS, RPS)])


def _gs_body(src_hbm, dst_hbm, z_hbm, zeros_hbm, out_hbm,
             sidx, didx, acc, *bufs):
    rows = bufs[:NB]
    gsem = bufs[NB:2 * NB]
    ssem = bufs[2 * NB:3 * NB]
    c = lax.axis_index("c")
    s = lax.axis_index("s")
    w = c * NS + s
    pltpu.sync_copy(zeros_hbm, acc.at[pl.ds(s * RPS, RPS)])
    pltpu.sync_copy(src_hbm.at[w], sidx)
    pltpu.sync_copy(dst_hbm.at[w], didx)
    plsc.subcore_barrier()

    def gather(k, b):
        pltpu.async_copy(z_hbm.at[sidx.at[k]], rows[b], gsem[b])

    def gather_wait(k, b):
        pltpu.make_async_copy(z_hbm.at[sidx.at[k]], rows[b], gsem[b]).wait()

    def scatter(k, b):
        pltpu.async_copy(rows[b], acc.at[didx.at[k]], ssem[b], add=True)

    def scatter_wait(k, b):
        pltpu.make_async_copy(rows[b], acc.at[didx.at[k]], ssem[b]).wait()

    GA = NB - 3

    for b in range(GA):
        gather(b, b)
    for b in range(NB):
        k = b
        gather_wait(k, b)
        scatter(k, b)
        bq = (b + GA) % NB
        if k >= 3:
            scatter_wait(k - 3, bq)
        gather(k + GA, bq)

    def block(j, carry):
        for b in range(NB):
            k = j * NB + b
            gather_wait(k, b)
            scatter(k, b)
            bq = (b + GA) % NB
            scatter_wait(k - 3, bq)
            gather(k + GA, bq)
        return carry

    nfull = CHUNKS // NB
    lax.fori_loop(1, nfull, block, 0)
    for k in range(nfull * NB, CHUNKS):
        b = k % NB
        gather_wait(k, b)
        scatter(k, b)
        scatter_wait(k - 3, (b + GA) % NB)
        if k + GA < CHUNKS:
            gather(k + GA, (b + GA) % NB)
    for k in range(CHUNKS - 3, CHUNKS):
        scatter_wait(k, k % NB)

    plsc.subcore_barrier()
    pltpu.sync_copy(acc.at[pl.ds(s * RPS, RPS)],
                    out_hbm.at[c, pl.ds(s * RPS, RPS)])


@functools.cache
def _sc_kernels():
    mesh = plsc.VectorSubcoreMesh(core_axis_name="c", subcore_axis_name="s",
                                  num_cores=NC, num_subcores=NS)
    sc_degree = functools.partial(
        pl.kernel,
        out_type=jax.ShapeDtypeStruct((NC, NP, DEGW), jnp.float32),
        mesh=mesh,
        compiler_params=pltpu.CompilerParams(use_tc_tiling_on_sc=False),
        scratch_types=[
            pltpu.VMEM((CHUNKS, C), jnp.int32),
            pltpu.VMEM((C, DEGW), jnp.float32),
            pltpu.VMEM_SHARED((NP, DEGW), jnp.float32),
            pltpu.SemaphoreType.DMA,
        ],
    )(_degree_body)
    sc_gather_scatter = functools.partial(
        pl.kernel,
        out_type=jax.ShapeDtypeStruct((NC, NP, D), jnp.float32),
        mesh=mesh,
        compiler_params=pltpu.CompilerParams(use_tc_tiling_on_sc=False),
        scratch_types=(
            [pltpu.VMEM((CHUNKS, C), jnp.int32),
             pltpu.VMEM((CHUNKS, C), jnp.int32),
             pltpu.VMEM_SHARED((NP, D), jnp.float32)]
            + [pltpu.VMEM((C, D), jnp.float32) for _ in range(NB)]
            + [pltpu.SemaphoreType.DMA for _ in range(2 * NB)]
        ),
    )(_gs_body)
    return sc_degree, sc_gather_scatter


R = 2000


def _bdot(a, b):
    return jnp.dot(a, b, preferred_element_type=jnp.float32)


def _hdot(a, b):
    return jnp.dot(a, b, preferred_element_type=jnp.float32,
                   precision=lax.Precision.HIGHEST)


def _dinv_block(deg_ref):
    deg = 1.0 + deg_ref[0, :, 0:1] + deg_ref[1, :, 0:1]
    return 1.0 / jnp.sqrt(deg)


def _mm1_kernel(nodes_ref, wfc_ref, bfc_ref, w1_ref, deg_ref, z1_ref):
    h = jnp.maximum(_bdot(nodes_ref[...], wfc_ref[...]) + bfc_ref[...], 0.0)
    z1_ref[...] = _bdot(h, w1_ref[...]) * _dinv_block(deg_ref)


def _mm2_kernel(a_ref, z1_ref, deg_ref, b1_ref, w2_ref, z2_ref):
    dinv = _dinv_block(deg_ref)
    h1 = jnp.maximum(
        (a_ref[0] + a_ref[1] + z1_ref[...]) * dinv + b1_ref[...], 0.0)
    z2_ref[...] = _bdot(h1, w2_ref[...]) * dinv


def _fin_kernel(a_ref, z2_ref, deg_ref, b2_ref, wa_ref, ba_ref, wc_ref,
                bc_ref, logits_ref, value_ref, sum_ref):
    i = pl.program_id(0)
    dinv = _dinv_block(deg_ref)
    h2 = jnp.maximum(
        (a_ref[0] + a_ref[1] + z2_ref[...]) * dinv + b2_ref[...], 0.0)
    part = jnp.sum(h2, axis=0, keepdims=True)

    @pl.when(i == 0)
    def _init():
        sum_ref[...] = jnp.zeros_like(sum_ref)

    sum_ref[...] += part

    @pl.when(i == pl.num_programs(0) - 1)
    def _finish():
        rep = sum_ref[...] * (1.0 / N)
        logits_ref[...] = _hdot(rep, wa_ref[...]) + ba_ref[...]
        value_ref[...] = _hdot(rep, wc_ref[...]) + bc_ref[...]


def _full(shape):
    return pl.BlockSpec(shape, lambda i: (0,) * len(shape))


@functools.cache
def _tc_kernels():
    mm1 = pl.pallas_call(
    _mm1_kernel,
    grid=(N // R,),
    in_specs=[
        pl.BlockSpec((R, D_FEAT), lambda i: (i, 0)),
        _full((D_FEAT, D)),
        _full((1, D)),
        _full((D, D)),
        pl.BlockSpec((NC, R, DEGW), lambda i: (0, i, 0)),
    ],
    out_specs=pl.BlockSpec((R, D), lambda i: (i, 0)),
    out_shape=jax.ShapeDtypeStruct((N, D), jnp.float32),
    )
    mm2 = pl.pallas_call(
    _mm2_kernel,
    grid=(N // R,),
    in_specs=[
        pl.BlockSpec((NC, R, D), lambda i: (0, i, 0)),
        pl.BlockSpec((R, D), lambda i: (i, 0)),
        pl.BlockSpec((NC, R, DEGW), lambda i: (0, i, 0)),
        _full((1, D)),
        _full((D, D)),
    ],
    out_specs=pl.BlockSpec((R, D), lambda i: (i, 0)),
    out_shape=jax.ShapeDtypeStruct((N, D), jnp.float32),
    )
    fin = pl.pallas_call(
    _fin_kernel,
    grid=(N // R,),
    in_specs=[
        pl.BlockSpec((NC, R, D), lambda i: (0, i, 0)),
        pl.BlockSpec((R, D), lambda i: (i, 0)),
        pl.BlockSpec((NC, R, DEGW), lambda i: (0, i, 0)),
        _full((1, D)),
        _full((D, NUM_ACTIONS)),
        _full((1, NUM_ACTIONS)),
        _full((D, 1)),
        _full((1, 1)),
    ],
    out_specs=[_full((1, NUM_ACTIONS)), _full((1, 1))],
    out_shape=[jax.ShapeDtypeStruct((1, NUM_ACTIONS), jnp.float32),
               jax.ShapeDtypeStruct((1, 1), jnp.float32)],
    scratch_shapes=[pltpu.VMEM((1, D), jnp.float32)],
    )
    return mm1, mm2, fin


def kernel(nodes, edge_links, W_fc, b_fc, W1, b1, W2, b2, Wa, ba, Wc, bc):
    src = edge_links[:, 0].reshape(NW, CHUNKS, C)
    dst = edge_links[:, 1].reshape(NW, CHUNKS, C)
    zeros_d = jnp.zeros((RPS, D), jnp.float32)
    zeros_g = jnp.zeros((RPS, DEGW), jnp.float32)
    ones_g = jnp.ones((C, DEGW), jnp.float32)

    sc_degree, sc_gather_scatter = _sc_kernels()
    mm1, mm2, fin = _tc_kernels()
    deg = sc_degree(dst, ones_g, zeros_g)
    z1 = mm1(nodes, W_fc, b_fc.reshape(1, D), W1, deg)
    a1 = sc_gather_scatter(src, dst, z1, zeros_d)
    z2 = mm2(a1, z1, deg, b1.reshape(1, D), W2)
    a2 = sc_gather_scatter(src, dst, z2, zeros_d)
    logits, value = fin(a2, z2, deg, b2.reshape(1, D), Wa,
                        ba.reshape(1, NUM_ACTIONS), Wc, bc.reshape(1, 1))
    return logits, value


# speedup vs baseline: 1.5905x; 1.0101x over previous
import functools

import jax
import jax.numpy as jnp
from jax import lax
from jax.experimental import pallas as pl
from jax.experimental.pallas import tpu as pltpu
from jax.experimental.pallas import tpu_sc as plsc

N = 10000
E = 320000
D_FEAT = 128
D = 64
NUM_ACTIONS = 10

NC = 2
NS = 16
NW = NC * NS
C = 80
EPW = E // NW
CHUNKS = EPW // C
NB = 12
NP = 10240
RPS = NP // NS
DEGW = 16


def _degree_body(dst_hbm, ones_hbm, zeros_hbm, out_hbm, didx, ones_v, acc, sem):
    c = lax.axis_index("c")
    s = lax.axis_index("s")
    w = c * NS + s
    pltpu.sync_copy(zeros_hbm, acc.at[pl.ds(s * RPS, RPS)])
    pltpu.sync_copy(dst_hbm.at[w], didx)
    pltpu.sync_copy(ones_hbm, ones_v)
    plsc.subcore_barrier()

    def fire(k, carry):
        pltpu.async_copy(ones_v, acc.at[didx.at[k]], sem, add=True)
        return carry

    lax.fori_loop(0, CHUNKS, fire, 0)

    def drain(k, carry):
        pltpu.make_async_copy(ones_v, acc.at[didx.at[k]], sem).wait()
        return carry

    lax.fori_loop(0, CHUNKS, drain, 0)
    plsc.subcore_barrier()
    pltpu.sync_copy(acc.at[pl.ds(s * RPS, RPS)],
                    out_hbm.at[c, pl.ds(s * RPS, RPS)])


def _gs_body(src_hbm, dst_hbm, z_hbm, zeros_hbm, out_hbm,
             sidx, didx, acc, *bufs):
    rows = bufs[:NB]
    gsem = bufs[NB:2 * NB]
    ssem = bufs[2 * NB:3 * NB]
    c = lax.axis_index("c")
    s = lax.axis_index("s")
    w = c * NS + s
    pltpu.sync_copy(zeros_hbm, acc.at[pl.ds(s * RPS, RPS)])
    pltpu.sync_copy(src_hbm.at[w], sidx)
    pltpu.sync_copy(dst_hbm.at[w], didx)
    plsc.subcore_barrier()

    def gather(k, b):
        pltpu.async_copy(z_hbm.at[sidx.at[k]], rows[b], gsem[b])

    def gather_wait(k, b):
        pltpu.make_async_copy(z_hbm.at[sidx.at[k]], rows[b], gsem[b]).wait()

    def scatter(k, b):
        pltpu.async_copy(rows[b], acc.at[didx.at[k]], ssem[b], add=True)

    def scatter_wait(k, b):
        pltpu.make_async_copy(rows[b], acc.at[didx.at[k]], ssem[b]).wait()

    GA = NB - 3

    for b in range(GA):
        gather(b, b)
    for b in range(NB):
        k = b
        gather_wait(k, b)
        scatter(k, b)
        bq = (b + GA) % NB
        if k >= 3:
            scatter_wait(k - 3, bq)
        gather(k + GA, bq)

    def block(j, carry):
        for b in range(NB):
            k = j * NB + b
            gather_wait(k, b)
            scatter(k, b)
            bq = (b + GA) % NB
            scatter_wait(k - 3, bq)
            gather(k + GA, bq)
        return carry

    nfull = CHUNKS // NB
    lax.fori_loop(1, nfull, block, 0)
    for k in range(nfull * NB, CHUNKS):
        b = k % NB
        gather_wait(k, b)
        scatter(k, b)
        scatter_wait(k - 3, (b + GA) % NB)
        if k + GA < CHUNKS:
            gather(k + GA, (b + GA) % NB)
    for k in range(CHUNKS - 3, CHUNKS):
        scatter_wait(k, k % NB)

    plsc.subcore_barrier()
    pltpu.sync_copy(acc.at[pl.ds(s * RPS, RPS)],
                    out_hbm.at[c, pl.ds(s * RPS, RPS)])


@functools.cache
def _sc_kernels():
    mesh = plsc.VectorSubcoreMesh(core_axis_name="c", subcore_axis_name="s",
                                  num_cores=NC, num_subcores=NS)
    sc_degree = functools.partial(
        pl.kernel,
        out_type=jax.ShapeDtypeStruct((NC, NP, DEGW), jnp.float32),
        mesh=mesh,
        compiler_params=pltpu.CompilerParams(use_tc_tiling_on_sc=False),
        scratch_types=[
            pltpu.VMEM((CHUNKS, C), jnp.int32),
            pltpu.VMEM((C, DEGW), jnp.float32),
            pltpu.VMEM_SHARED((NP, DEGW), jnp.float32),
            pltpu.SemaphoreType.DMA,
        ],
    )(_degree_body)
    sc_gather_scatter = functools.partial(
        pl.kernel,
        out_type=jax.ShapeDtypeStruct((NC, NP, D), jnp.float32),
        mesh=mesh,
        compiler_params=pltpu.CompilerParams(use_tc_tiling_on_sc=False),
        scratch_types=(
            [pltpu.VMEM((CHUNKS, C), jnp.int32),
             pltpu.VMEM((CHUNKS, C), jnp.int32),
             pltpu.VMEM_SHARED((NP, D), jnp.float32)]
            + [pltpu.VMEM((C, D), jnp.float32) for _ in range(NB)]
            + [pltpu.SemaphoreType.DMA for _ in range(2 * NB)]
        ),
    )(_gs_body)
    return sc_degree, sc_gather_scatter


R = 2000


def _bdot(a, b):
    return jnp.dot(a, b, preferred_element_type=jnp.float32)


def _hdot(a, b):
    return jnp.dot(a, b, preferred_element_type=jnp.float32,
                   precision=lax.Precision.HIGHEST)


def _dinv_block(deg_ref):
    deg = 1.0 + deg_ref[0, :, 0:1] + deg_ref[1, :, 0:1]
    return 1.0 / jnp.sqrt(deg)


def _mm1_kernel(nodes_ref, wfc_ref, bfc_ref, w1_ref, deg_ref, z1_ref):
    h = jnp.maximum(_bdot(nodes_ref[...], wfc_ref[...]) + bfc_ref[...], 0.0)
    z1_ref[...] = _bdot(h, w1_ref[...]) * _dinv_block(deg_ref)


def _mm2_kernel(a_ref, z1_ref, deg_ref, b1_ref, w2_ref, z2_ref):
    dinv = _dinv_block(deg_ref)
    h1 = jnp.maximum(
        (a_ref[0] + a_ref[1] + z1_ref[...]) * dinv + b1_ref[...], 0.0)
    z2_ref[...] = _bdot(h1, w2_ref[...]) * dinv


def _fin_kernel(a_ref, z2_ref, deg_ref, b2_ref, wa_ref, ba_ref, wc_ref,
                bc_ref, logits_ref, value_ref, sum_ref):
    i = pl.program_id(0)
    dinv = _dinv_block(deg_ref)
    h2 = jnp.maximum(
        (a_ref[0] + a_ref[1] + z2_ref[...]) * dinv + b2_ref[...], 0.0)
    part = jnp.sum(h2, axis=0, keepdims=True)

    @pl.when(i == 0)
    def _init():
        sum_ref[...] = jnp.zeros_like(sum_ref)

    sum_ref[...] += part

    @pl.when(i == pl.num_programs(0) - 1)
    def _finish():
        rep = sum_ref[...] * (1.0 / N)
        logits_ref[...] = _hdot(rep, wa_ref[...]) + ba_ref[...]
        value_ref[...] = _hdot(rep, wc_ref[...]) + bc_ref[...]


def _full(shape):
    return pl.BlockSpec(shape, lambda i: (0,) * len(shape))


@functools.cache
def _tc_kernels():
    mm1 = pl.pallas_call(
    _mm1_kernel,
    grid=(N // R,),
    in_specs=[
        pl.BlockSpec((R, D_FEAT), lambda i: (i, 0)),
        _full((D_FEAT, D)),
        _full((1, D)),
        _full((D, D)),
        pl.BlockSpec((NC, R, DEGW), lambda i: (0, i, 0)),
    ],
    out_specs=pl.BlockSpec((R, D), lambda i: (i, 0)),
    out_shape=jax.ShapeDtypeStruct((N, D), jnp.float32),
    )
    mm2 = pl.pallas_call(
    _mm2_kernel,
    grid=(N // R,),
    in_specs=[
        pl.BlockSpec((NC, R, D), lambda i: (0, i, 0)),
        pl.BlockSpec((R, D), lambda i: (i, 0)),
        pl.BlockSpec((NC, R, DEGW), lambda i: (0, i, 0)),
        _full((1, D)),
        _full((D, D)),
    ],
    out_specs=pl.BlockSpec((R, D), lambda i: (i, 0)),
    out_shape=jax.ShapeDtypeStruct((N, D), jnp.float32),
    )
    fin = pl.pallas_call(
    _fin_kernel,
    grid=(N // R,),
    in_specs=[
        pl.BlockSpec((NC, R, D), lambda i: (0, i, 0)),
        pl.BlockSpec((R, D), lambda i: (i, 0)),
        pl.BlockSpec((NC, R, DEGW), lambda i: (0, i, 0)),
        _full((1, D)),
        _full((D, NUM_ACTIONS)),
        _full((1, NUM_ACTIONS)),
        _full((D, 1)),
        _full((1, 1)),
    ],
    out_specs=[_full((1, NUM_ACTIONS)), _full((1, 1))],
    out_shape=[jax.ShapeDtypeStruct((1, NUM_ACTIONS), jnp.float32),
               jax.ShapeDtypeStruct((1, 1), jnp.float32)],
    scratch_shapes=[pltpu.VMEM((1, D), jnp.float32)],
    )
    return mm1, mm2, fin


def kernel(nodes, edge_links, W_fc, b_fc, W1, b1, W2, b2, Wa, ba, Wc, bc):
    src = edge_links[:, 0].reshape(NW, CHUNKS, C)
    dst = edge_links[:, 1].reshape(NW, CHUNKS, C)
    zeros_d = jnp.zeros((RPS, D), jnp.float32)
    zeros_g = jnp.zeros((RPS, DEGW), jnp.float32)
    ones_g = jnp.ones((C, DEGW), jnp.float32)

    sc_degree, sc_gather_scatter = _sc_kernels()
    mm1, mm2, fin = _tc_kernels()
    deg = sc_degree(dst, ones_g, zeros_g)
    z1 = mm1(nodes, W_fc, b_fc.reshape(1, D), W1, deg)
    a1 = sc_gather_scatter(src, dst, z1, zeros_d)
    z2 = mm2(a1, z1, deg, b1.reshape(1, D), W2)
    a2 = sc_gather_scatter(src, dst, z2, zeros_d)
    logits, value = fin(a2, z2, deg, b2.reshape(1, D), Wa,
                        ba.reshape(1, NUM_ACTIONS), Wc, bc.reshape(1, 1))
    return logits, value
